# Initial kernel scaffold; baseline (speedup 1.0000x reference)
import jax
import jax.numpy as jnp
from jax.experimental import pallas as pl


def kernel(inputs_flatten, dictionary):
    raise NotImplementedError("write your pallas kernel here")


# speedup vs baseline: 22.6346x; 22.6346x over previous
import functools

import jax
import jax.numpy as jnp
from jax.experimental import pallas as pl

_N = 1024
_D = 256
_K = 4
_BB = 256


def _tc_body(x_ref, d_ref, idx_ref, emb_ref):
    x = x_ref[...]
    d = d_ref[...]
    xsq = jnp.sum(x * x, axis=1, keepdims=True)
    dsq = jnp.sum(d * d, axis=1)[None, :]
    xd = jax.lax.dot_general(
        x, d, (((1,), (1,)), ((), ())),
        preferred_element_type=jnp.float32)
    dist = xsq + dsq - 2.0 * xd

    iota = jax.lax.broadcasted_iota(jnp.int32, (_BB, _N), 1)
    work = dist
    onehot = jnp.zeros((_BB, _N), jnp.float32)
    cols = []
    for _ in range(_K):
        m = jnp.min(work, axis=1, keepdims=True)
        idx_k = jnp.min(jnp.where(work == m, iota, _N), axis=1)
        sel = iota == idx_k[:, None]
        onehot = onehot + sel.astype(jnp.float32)
        work = jnp.where(sel, jnp.inf, work)
        cols.append(idx_k[:, None])
    idx_ref[...] = jnp.concatenate(cols, axis=1)

    g = jax.lax.dot_general(
        onehot, d, (((1,), (0,)), ((), ())),
        preferred_element_type=jnp.float32) * (1.0 / _K)
    emb_ref[...] = (g - x) + x


def kernel(inputs_flatten, dictionary):
    b = inputs_flatten.shape[0]
    grid = (b // _BB,)
    idx, emb = pl.pallas_call(
        _tc_body,
        grid=grid,
        in_specs=[
            pl.BlockSpec((_BB, _D), lambda i: (i, 0)),
            pl.BlockSpec((_N, _D), lambda i: (0, 0)),
        ],
        out_specs=[
            pl.BlockSpec((_BB, _K), lambda i: (i, 0)),
            pl.BlockSpec((_BB, _D), lambda i: (i, 0)),
        ],
        out_shape=[
            jax.ShapeDtypeStruct((b, _K), jnp.int32),
            jax.ShapeDtypeStruct((b, _D), jnp.float32),
        ],
    )(inputs_flatten, dictionary)
    return (emb, idx)
